# Initial kernel scaffold; baseline (speedup 1.0000x reference)
import jax
import jax.numpy as jnp
from jax.experimental import pallas as pl


def kernel(x, edge_index_0, edge_index_1, W_l1, b_l1, W_r1, W_l2, b_l2, W_r2):
    raise NotImplementedError("write your pallas kernel here")


# speedup vs baseline: 1.4345x; 1.4345x over previous
import functools

import jax
import jax.numpy as jnp
from jax import lax
from jax.experimental import pallas as pl
from jax.experimental.pallas import tpu as pltpu
from jax.experimental.pallas import tpu_sc as plsc

N0, N1, N2 = 100000, 20000, 2048
E1, E2 = 320000, 32768
D_IN, D_H, N_CLS = 128, 128, 47

B = 64
ROWS1 = 5120
E1P = ROWS1 * B
RPT1 = ROWS1 // 32
CH1 = 16
QR = 4096
N1A = 4224
ZR1 = N1A // 16
NQ = 5
N1G = NQ * QR

ROWS2 = E2 // B
RPT2 = ROWS2 // 32
ZR2 = N2 // 16

_vmesh = plsc.VectorSubcoreMesh(core_axis_name="c", subcore_axis_name="s")


def _zero_acc(bufa, acc, t, zr):
    nfull = zr // B

    @pl.loop(0, nfull)
    def _(i):
        pltpu.sync_copy(bufa, acc.at[pl.ds(t * zr + i * B, B)])

    rem = zr - nfull * B
    if rem:
        pltpu.sync_copy(bufa.at[pl.ds(0, rem)],
                        acc.at[pl.ds(t * zr + nfull * B, rem)])


def _write_acc(acc, bufa, out_hbm, c, t, zr, na):
    nfull = zr // B

    @pl.loop(0, nfull)
    def _(i):
        pltpu.sync_copy(acc.at[pl.ds(t * zr + i * B, B)], bufa)
        pltpu.sync_copy(bufa, out_hbm.at[pl.ds(c * na + t * zr + i * B, B)])

    rem = zr - nfull * B
    if rem:
        pltpu.sync_copy(acc.at[pl.ds(t * zr + nfull * B, rem)],
                        bufa.at[pl.ds(0, rem)])
        pltpu.sync_copy(bufa.at[pl.ds(0, rem)],
                        out_hbm.at[pl.ds(c * na + t * zr + nfull * B, rem)])


def _remap(tgtv, nrows, lo, hrange, tmask):
    @pl.loop(0, nrows)
    def _(r):
        @pl.loop(0, B // 16)
        def _(q):
            v = tgtv[r, pl.ds(q * 16, 16)]
            loc = v - lo
            oob = (loc < 0) | (loc >= hrange)
            tgtv[r, pl.ds(q * 16, 16)] = jnp.where(
                oob, hrange + (v & tmask), loc)


def _sc_layer1(x, src, tgt, z128, phase):
    @functools.partial(
        pl.kernel,
        out_type=jax.ShapeDtypeStruct((2 * N1A, D_IN), jnp.float32),
        mesh=_vmesh,
        scratch_types=[
            pltpu.VMEM((CH1, B), jnp.int32),
            pltpu.VMEM((CH1, B), jnp.int32),
            pltpu.VMEM((B, D_IN), jnp.float32),
            pltpu.VMEM((B, D_IN), jnp.float32),
            pltpu.VMEM_SHARED((N1A, D_IN), jnp.float32),
            pltpu.SemaphoreType.DMA,
        ],
    )
    def k(x_hbm, src_hbm, tgt_hbm, z128_hbm,
          sums_hbm, srcv, tgtv, bufa, bufb, acc, sema):
        c = lax.axis_index("c")
        t = lax.axis_index("s")

        pltpu.sync_copy(z128_hbm.at[pl.ds(0, B)], bufa)
        _zero_acc(bufa, acc, t, ZR1)
        plsc.subcore_barrier()

        base = c * (16 * RPT1) + t * RPT1
        lo = phase * QR

        @pl.loop(0, RPT1 // CH1)
        def _(g):
            cb = base + g * CH1
            pltpu.sync_copy(src_hbm.at[pl.ds(cb, CH1)], srcv)
            pltpu.sync_copy(tgt_hbm.at[pl.ds(cb, CH1)], tgtv)
            _remap(tgtv, CH1, lo, QR, 127)

            for i in range(CH1):
                pltpu.async_copy(x_hbm.at[srcv.at[i]], bufb, sema).wait()
                pltpu.sync_copy(bufb, acc.at[tgtv.at[i]], add=True)

        plsc.subcore_barrier()
        _write_acc(acc, bufa, sums_hbm, c, t, ZR1, N1A)

    return k(x, src, tgt, z128)


def _sc_count1(tgt, z128, ones128, phase):
    @functools.partial(
        pl.kernel,
        out_type=jax.ShapeDtypeStruct((2 * N1A, D_IN), jnp.float32),
        mesh=_vmesh,
        scratch_types=[
            pltpu.VMEM((CH1, B), jnp.int32),
            pltpu.VMEM((B, D_IN), jnp.float32),
            pltpu.VMEM((B, D_IN), jnp.float32),
            pltpu.VMEM_SHARED((N1A, D_IN), jnp.float32),
        ],
    )
    def k(tgt_hbm, z128_hbm, ones_hbm, cnt_hbm, tgtv, bufa, onesv, acc):
        c = lax.axis_index("c")
        t = lax.axis_index("s")

        pltpu.sync_copy(z128_hbm.at[pl.ds(0, B)], bufa)
        _zero_acc(bufa, acc, t, ZR1)
        pltpu.sync_copy(ones_hbm, onesv)
        plsc.subcore_barrier()

        base = c * (16 * RPT1) + t * RPT1
        lo = phase * QR

        @pl.loop(0, RPT1 // CH1)
        def _(g):
            cb = base + g * CH1
            pltpu.sync_copy(tgt_hbm.at[pl.ds(cb, CH1)], tgtv)
            _remap(tgtv, CH1, lo, QR, 127)

            for i in range(CH1):
                pltpu.sync_copy(onesv, acc.at[tgtv.at[i]], add=True)

        plsc.subcore_barrier()
        _write_acc(acc, bufa, cnt_hbm, c, t, ZR1, N1A)

    return k(tgt, z128, ones128)


def _sc_layer2(h, src, tgt, z128, ones):
    @functools.partial(
        pl.kernel,
        out_type=[
            jax.ShapeDtypeStruct((2 * N2, D_H), jnp.float32),
            jax.ShapeDtypeStruct((2 * N2, D_H), jnp.float32),
        ],
        mesh=_vmesh,
        scratch_types=[
            pltpu.VMEM((RPT2, B), jnp.int32),
            pltpu.VMEM((RPT2, B), jnp.int32),
            pltpu.VMEM((B, D_H), jnp.float32),
            pltpu.VMEM((B, D_H), jnp.float32),
            pltpu.VMEM((B, D_H), jnp.float32),
            pltpu.VMEM_SHARED((N2, D_H), jnp.float32),
            pltpu.VMEM_SHARED((N2, D_H), jnp.float32),
            pltpu.SemaphoreType.DMA,
        ],
    )
    def k(h_hbm, src_hbm, tgt_hbm, z128_hbm, ones_hbm,
          sums_hbm, cnt_hbm,
          srcv, tgtv, bufa, bufb, onesv, acc, cacc, sema):
        c = lax.axis_index("c")
        t = lax.axis_index("s")

        pltpu.sync_copy(z128_hbm.at[pl.ds(0, B)], bufa)
        _zero_acc(bufa, acc, t, ZR2)
        _zero_acc(bufa, cacc, t, ZR2)
        base = c * (16 * RPT2) + t * RPT2
        pltpu.sync_copy(src_hbm.at[pl.ds(base, RPT2)], srcv)
        pltpu.sync_copy(tgt_hbm.at[pl.ds(base, RPT2)], tgtv)
        pltpu.sync_copy(ones_hbm, onesv)
        plsc.subcore_barrier()

        for i in range(RPT2):
            pltpu.async_copy(h_hbm.at[srcv.at[i]], bufb, sema).wait()
            pltpu.sync_copy(bufb, acc.at[tgtv.at[i]], add=True)
            pltpu.sync_copy(onesv, cacc.at[tgtv.at[i]], add=True)

        plsc.subcore_barrier()
        _write_acc(acc, bufa, sums_hbm, c, t, ZR2, N2)
        _write_acc(cacc, bufa, cnt_hbm, c, t, ZR2, N2)

    return k(h, src, tgt, z128, ones)


def _dense1_body(s_ref, cp_ref, x_ref, wl_ref, wr_ref, b_ref, h_ref):
    cnt = jnp.maximum(cp_ref[0, 0, :, 0:1] + cp_ref[0, 1, :, 0:1], 1.0)
    mean = (s_ref[0, 0] + s_ref[0, 1]) / cnt
    h = (jnp.dot(mean, wl_ref[...], preferred_element_type=jnp.float32)
         + jnp.dot(x_ref[...], wr_ref[...], preferred_element_type=jnp.float32)
         + b_ref[...])
    h_ref[...] = jnp.maximum(h, 0.0)


def _dense2_body(s_ref, cp_ref, h_ref, wl_ref, wr_ref, b_ref, o_ref):
    cnt = jnp.maximum(cp_ref[0, :, 0:1] + cp_ref[1, :, 0:1], 1.0)
    mean = (s_ref[0] + s_ref[1]) / cnt
    logits = (jnp.dot(mean, wl_ref[...], preferred_element_type=jnp.float32)
              + jnp.dot(h_ref[...], wr_ref[...],
                        preferred_element_type=jnp.float32)
              + b_ref[...])
    mx = jnp.max(logits, axis=1, keepdims=True)
    lse = jnp.log(jnp.sum(jnp.exp(logits - mx), axis=1, keepdims=True))
    o_ref[...] = logits - mx - lse


def kernel(x, edge_index_0, edge_index_1, W_l1, b_l1, W_r1, W_l2, b_l2, W_r2):
    f32 = jnp.float32
    pad = E1P - E1
    src1 = jnp.concatenate(
        [edge_index_0[0].astype(jnp.int32),
         (jnp.arange(pad, dtype=jnp.int32) * 131) % N0]).reshape(ROWS1, B)
    tgt1 = jnp.concatenate(
        [edge_index_0[1].astype(jnp.int32),
         N1 + (jnp.arange(pad, dtype=jnp.int32) % 96)]).reshape(ROWS1, B)
    src2 = edge_index_1[0].astype(jnp.int32).reshape(ROWS2, B)
    tgt2 = edge_index_1[1].astype(jnp.int32).reshape(ROWS2, B)

    z128 = jnp.zeros((N1A, D_IN), f32)
    ones = jnp.ones((B, D_IN), f32)

    sums_parts, cnt_parts = [], []
    for q in range(NQ):
        s = _sc_layer1(x, src1, tgt1, z128, q)
        cc = _sc_count1(tgt1, z128, ones, q)
        sums_parts.append(s.reshape(2, N1A, D_IN))
        cnt_parts.append(cc.reshape(2, N1A, D_IN))
    sums1 = jnp.stack(sums_parts)
    cnt1 = jnp.stack(cnt_parts)

    R = 1024
    h = pl.pallas_call(
        _dense1_body,
        grid=(N1G // R,),
        in_specs=[
            pl.BlockSpec((1, 2, R, D_IN), lambda i: (i // 4, 0, i % 4, 0)),
            pl.BlockSpec((1, 2, R, D_IN), lambda i: (i // 4, 0, i % 4, 0)),
            pl.BlockSpec((R, D_IN), lambda i: (i, 0)),
            pl.BlockSpec((D_IN, D_H), lambda i: (0, 0)),
            pl.BlockSpec((D_IN, D_H), lambda i: (0, 0)),
            pl.BlockSpec((1, D_H), lambda i: (0, 0)),
        ],
        out_specs=pl.BlockSpec((R, D_H), lambda i: (i, 0)),
        out_shape=jax.ShapeDtypeStruct((N1G, D_H), f32),
    )(sums1, cnt1, x, W_l1, W_r1, b_l1.reshape(1, D_H))

    z128b = jnp.zeros((N2, D_H), f32)
    sums2, cnt2 = _sc_layer2(h, src2, tgt2, z128b, ones)
    sums2 = sums2.reshape(2, N2, D_H)
    cnt2 = cnt2.reshape(2, N2, D_H)

    out = pl.pallas_call(
        _dense2_body,
        grid=(1,),
        in_specs=[
            pl.BlockSpec((2, N2, D_H), lambda i: (0, 0, 0)),
            pl.BlockSpec((2, N2, D_H), lambda i: (0, 0, 0)),
            pl.BlockSpec((N2, D_H), lambda i: (0, 0)),
            pl.BlockSpec((D_H, N_CLS), lambda i: (0, 0)),
            pl.BlockSpec((D_H, N_CLS), lambda i: (0, 0)),
            pl.BlockSpec((1, N_CLS), lambda i: (0, 0)),
        ],
        out_specs=pl.BlockSpec((N2, N_CLS), lambda i: (0, 0)),
        out_shape=jax.ShapeDtypeStruct((N2, N_CLS), f32),
    )(sums2, cnt2, h, W_l2, W_r2, b_l2.reshape(1, N_CLS))

    return out
